# baseline (speedup 1.0000x reference)
import functools

import jax
import jax.numpy as jnp
from jax import lax
from jax.experimental import pallas as pl
from jax.experimental.pallas import tpu as pltpu
from jax.experimental.pallas import tpu_sc as plsc

_D = 32
_NC = 2
_NS = 16
_NW = _NC * _NS
_H = 50
_B = 16384
_BPW = _B // _NW


@functools.lru_cache(maxsize=None)
def _make_gather():
    mesh = plsc.VectorSubcoreMesh(core_axis_name="c", subcore_axis_name="s")

    @functools.partial(
        pl.kernel,
        mesh=mesh,
        out_type=jax.ShapeDtypeStruct((_H, _B, _D), jnp.float32),
        scratch_types=[
            pltpu.VMEM((_H, _BPW), jnp.int32),
            pltpu.VMEM((4, _BPW, _D), jnp.float32),
        ]
        + [pltpu.SemaphoreType.DMA] * 4,
        compiler_params=pltpu.CompilerParams(
            use_tc_tiling_on_sc=False, needs_layout_passes=False
        ),
    )
    def k(table_hbm, idx_hbm, out_hbm, idx_v, rows_v, gs0, gs1, os0, os1):
        gsems = (gs0, gs1)
        osems = (os0, os1)
        wid = lax.axis_index("s") * _NC + lax.axis_index("c")
        b0 = wid * _BPW

        pltpu.sync_copy(idx_hbm.at[:, pl.ds(b0, _BPW)], idx_v)

        def gather(h, buf):
            return pltpu.make_async_copy(
                table_hbm.at[idx_v.at[h]], rows_v.at[buf], gsems[buf % 2]
            )

        def outcp(h, buf):
            return pltpu.make_async_copy(
                rows_v.at[buf],
                out_hbm.at[h, pl.ds(b0, _BPW)],
                osems[buf % 2],
            )

        gather(0, 0).start()
        gather(1, 1).start()

        def step(h, buf):
            gather(h, buf).wait()
            outcp(h, buf).start()

            @pl.when(h >= 2)
            def _():
                outcp(h - 2, (buf + 2) % 4).wait()

            @pl.when(h + 2 < _H)
            def _():
                gather(h + 2, (buf + 2) % 4).start()

        def body(o, carry):
            h0 = 4 * o
            for j in range(4):
                step(h0 + j, j)
            return carry

        lax.fori_loop(0, (_H - 2) // 4, body, 0)

        for h in range(((_H - 2) // 4) * 4, _H):
            gather(h, h % 4).wait()
            outcp(h, h % 4).start()
            outcp(h - 2, (h - 2) % 4).wait()

        outcp(_H - 2, (_H - 2) % 4).wait()
        outcp(_H - 1, (_H - 1) % 4).wait()

    return k


@jax.jit
def _run(idx_t, weight):
    return _make_gather()(weight, idx_t)


def kernel(input, weight):
    idx_t = input.astype(jnp.int32).T.reshape(_H, _B)
    o_hbd = _run(idx_t, weight)
    return o_hbd.transpose(1, 0, 2)
